# baseline (speedup 1.0000x reference)
import functools

import jax
import jax.numpy as jnp
from jax import lax
from jax.experimental import pallas as pl
from jax.experimental.pallas import tpu as pltpu
from jax.experimental.pallas import tpu_sc as plsc

NC = 2
NS = 16
B = 88
DEPTH = 3
RING = 2 * DEPTH


def _seg_sum_call(table, src_p, dst_p, zeros_b, n_pad, nb0, nb1):
    D = table.shape[1]
    rpt = n_pad // NS

    mesh = plsc.VectorSubcoreMesh(core_axis_name="c", subcore_axis_name="s")

    @functools.partial(
        pl.kernel,
        out_type=jax.ShapeDtypeStruct((NC, n_pad, D), jnp.float32),
        mesh=mesh,
        scratch_types=[
            pltpu.VMEM_SHARED((n_pad, D), jnp.float32),
            pltpu.VMEM((RING, B), jnp.int32),
            pltpu.VMEM((RING, B), jnp.int32),
            pltpu.VMEM((DEPTH, B, D), jnp.float32),
            [pltpu.SemaphoreType.DMA] * DEPTH,
            [pltpu.SemaphoreType.DMA] * RING,
        ],
    )
    def seg_sum(table_hbm, src_hbm, dst_hbm, zero_hbm, out_hbm,
                acc, srcb, dstb, rows_v, gsem, semi):
        c = lax.axis_index("c")
        s = lax.axis_index("s")
        nbc = jnp.where(c == 0, nb0, nb1)

        def idx_start(batch, k):
            pltpu.async_copy(src_hbm.at[c, s, batch], srcb.at[k], semi[k])
            pltpu.async_copy(dst_hbm.at[c, s, batch], dstb.at[k], semi[k])

        def idx_wait(k):
            pltpu.make_async_copy(src_hbm.at[0, 0, 0], srcb.at[k], semi[k]).wait()
            pltpu.make_async_copy(src_hbm.at[0, 0, 0], dstb.at[k], semi[k]).wait()

        def gather_start(k, p):
            pltpu.async_copy(table_hbm.at[srcb.at[k]], rows_v.at[p], gsem[p])

        def gather_wait(p):
            pltpu.make_async_copy(
                table_hbm.at[srcb.at[0]], rows_v.at[p], gsem[p]).wait()

        pltpu.sync_copy(zero_hbm, rows_v.at[0])
        base = s * rpt
        nfull, tail = rpt // B, rpt % B
        for k in range(nfull):
            pltpu.sync_copy(rows_v.at[0], acc.at[pl.ds(base + k * B, B)])
        if tail:
            pltpu.sync_copy(rows_v.at[0, pl.ds(0, tail)],
                            acc.at[pl.ds(base + nfull * B, tail)])

        for k in range(RING):
            idx_start(k, k)
        for k in range(DEPTH):
            idx_wait(k)
        plsc.subcore_barrier()
        for k in range(DEPTH):
            gather_start(k, k)

        def body(g, carry):
            for k in range(RING):
                j = g * RING + k
                p = k % DEPTH
                kpd = (k + DEPTH) % RING
                gather_wait(p)
                pltpu.sync_copy(rows_v.at[p], acc.at[dstb.at[k]], add=True)
                idx_start(jnp.minimum(j + RING, nbc - 1), k)
                idx_wait(kpd)
                gather_start(kpd, p)
            return carry

        lax.fori_loop(0, nbc // RING, body, 0)

        for k in range(DEPTH):
            idx_wait((DEPTH + k) % RING)
            gather_wait(k)

        plsc.subcore_barrier()
        pltpu.sync_copy(acc.at[pl.ds(s * rpt, rpt)],
                        out_hbm.at[c, pl.ds(s * rpt, rpt)])

    return seg_sum(table, src_p, dst_p, zeros_b)


def _mlp_call(h, agg, Wa, ba, Wb, bb):
    n, D = h.shape
    blk = 2000
    grid = (n // blk,)

    def body(h_ref, a_ref, wa_ref, ba_ref, wb_ref, bb_ref, o_ref):
        r = h_ref[...] + a_ref[0] + a_ref[1]
        t = jnp.maximum(
            jnp.dot(r, wa_ref[...], preferred_element_type=jnp.float32)
            + ba_ref[...], 0.0)
        o_ref[...] = (jnp.dot(t, wb_ref[...], preferred_element_type=jnp.float32)
                      + bb_ref[...])

    return pl.pallas_call(
        body,
        grid=grid,
        in_specs=[
            pl.BlockSpec((blk, D), lambda i: (i, 0)),
            pl.BlockSpec((2, blk, D), lambda i: (0, i, 0)),
            pl.BlockSpec((D, D), lambda i: (0, 0)),
            pl.BlockSpec((1, D), lambda i: (0, 0)),
            pl.BlockSpec((D, D), lambda i: (0, 0)),
            pl.BlockSpec((1, D), lambda i: (0, 0)),
        ],
        out_specs=pl.BlockSpec((blk, D), lambda i: (i, 0)),
        out_shape=jax.ShapeDtypeStruct((n, D), jnp.float32),
    )(h, agg, Wa, ba, Wb, bb)


def _final_call(h, agg, W3, b3, W4, b4, Wl, bl):
    n, D = h.shape
    O = Wl.shape[1]
    blk = 2000
    grid = (n // blk,)

    def body(h_ref, a_ref, w3_ref, b3_ref, w4_ref, b4_ref, wl_ref, bl_ref,
             o_ref, acc_ref):
        i = pl.program_id(0)

        @pl.when(i == 0)
        def _():
            acc_ref[...] = jnp.zeros_like(acc_ref)

        r = h_ref[...] + a_ref[0] + a_ref[1]
        t = jnp.maximum(
            jnp.dot(r, w3_ref[...], preferred_element_type=jnp.float32)
            + b3_ref[...], 0.0)
        acc_ref[...] += jnp.sum(t, axis=0, keepdims=True)

        @pl.when(i == grid[0] - 1)
        def _():
            hg = acc_ref[...] * (1.0 / n)
            t2 = (jnp.dot(hg, w4_ref[...], preferred_element_type=jnp.float32)
                  + b4_ref[...])
            o_ref[...] = (jnp.dot(t2, wl_ref[...],
                                  preferred_element_type=jnp.float32)
                          + bl_ref[...])

    return pl.pallas_call(
        body,
        grid=grid,
        in_specs=[
            pl.BlockSpec((blk, D), lambda i: (i, 0)),
            pl.BlockSpec((2, blk, D), lambda i: (0, i, 0)),
            pl.BlockSpec((D, D), lambda i: (0, 0)),
            pl.BlockSpec((1, D), lambda i: (0, 0)),
            pl.BlockSpec((D, D), lambda i: (0, 0)),
            pl.BlockSpec((1, D), lambda i: (0, 0)),
            pl.BlockSpec((D, O), lambda i: (0, 0)),
            pl.BlockSpec((1, O), lambda i: (0, 0)),
        ],
        out_specs=pl.BlockSpec((1, O), lambda i: (0, 0)),
        out_shape=jax.ShapeDtypeStruct((1, O), jnp.float32),
        scratch_shapes=[pltpu.VMEM((1, D), jnp.float32)],
    )(h, agg, W3, b3, W4, b4, Wl, bl)


def kernel(features, edge_index, W1, b1, W2, b2, W3, b3, W4, b4, Wl, bl):
    n, D = features.shape
    E = edge_index.shape[1]

    n_pad = ((n + 1 + NS * 8 - 1) // (NS * 8)) * (NS * 8)
    per_b = NS * B
    nbt = -(-E // per_b)
    nbt = ((nbt + 2 * RING - 1) // (2 * RING)) * (2 * RING)
    nb0 = ((nbt * 39 // 100) // RING) * RING
    nb1 = nbt - nb0
    nbw = max(nb0, nb1)
    e_pad = nbt * per_b

    src = edge_index[0]
    dst = edge_index[1]
    pad = e_pad - E
    src_f = jnp.concatenate([src, jnp.zeros((pad,), jnp.int32)])
    dst_f = jnp.concatenate([dst, jnp.full((pad,), n, jnp.int32)])

    def split(flat):
        a = flat[:nb0 * per_b].reshape(NS, nb0, B)
        b = flat[nb0 * per_b:].reshape(NS, nb1, B)
        a = jnp.pad(a, ((0, 0), (0, nbw - nb0), (0, 0)))
        b = jnp.pad(b, ((0, 0), (0, nbw - nb1), (0, 0)))
        return jnp.stack([a, b])

    src_p = split(src_f)
    dst_p = split(dst_f)
    zeros_b = jnp.zeros((B, D), jnp.float32)

    b1r = b1.reshape(1, -1)
    b2r = b2.reshape(1, -1)
    b3r = b3.reshape(1, -1)
    b4r = b4.reshape(1, -1)
    blr = bl.reshape(1, -1)

    agg1 = _seg_sum_call(features, src_p, dst_p, zeros_b, n_pad, nb0, nb1)
    h1 = _mlp_call(features, agg1, W1, b1r, W2, b2r)
    agg2 = _seg_sum_call(h1, src_p, dst_p, zeros_b, n_pad, nb0, nb1)
    return _final_call(h1, agg2, W3, b3r, W4, b4r, Wl, blr)


# speedup vs baseline: 1.1753x; 1.1753x over previous
import functools

import jax
import jax.numpy as jnp
from jax import lax
from jax.experimental import pallas as pl
from jax.experimental.pallas import tpu as pltpu
from jax.experimental.pallas import tpu_sc as plsc

NC = 2
NS = 16
B = 88
DEPTH = 3
RING = 2 * DEPTH


def _seg_sum_call(table, src_p, dst_p, zeros_b, n_pad, nb0, nb1):
    D = table.shape[1]
    rpt = n_pad // NS

    mesh = plsc.VectorSubcoreMesh(core_axis_name="c", subcore_axis_name="s")

    @functools.partial(
        pl.kernel,
        out_type=jax.ShapeDtypeStruct((NC, n_pad, D), jnp.float32),
        mesh=mesh,
        scratch_types=[
            pltpu.VMEM_SHARED((n_pad, D), jnp.float32),
            pltpu.VMEM((RING, B), jnp.int32),
            pltpu.VMEM((RING, B), jnp.int32),
            pltpu.VMEM((DEPTH, B, D), jnp.float32),
            [pltpu.SemaphoreType.DMA] * DEPTH,
            [pltpu.SemaphoreType.DMA] * RING,
        ],
    )
    def seg_sum(table_hbm, src_hbm, dst_hbm, zero_hbm, out_hbm,
                acc, srcb, dstb, rows_v, gsem, semi):
        c = lax.axis_index("c")
        s = lax.axis_index("s")
        nbc = jnp.where(c == 0, nb0, nb1)

        def idx_start(batch, k):
            pltpu.async_copy(src_hbm.at[c, s, batch], srcb.at[k], semi[k])
            pltpu.async_copy(dst_hbm.at[c, s, batch], dstb.at[k], semi[k])

        def idx_wait(k):
            pltpu.make_async_copy(src_hbm.at[0, 0, 0], srcb.at[k], semi[k]).wait()
            pltpu.make_async_copy(src_hbm.at[0, 0, 0], dstb.at[k], semi[k]).wait()

        def gather_start(k, p):
            pltpu.async_copy(table_hbm.at[srcb.at[k]], rows_v.at[p], gsem[p])

        def gather_wait(p):
            pltpu.make_async_copy(
                table_hbm.at[srcb.at[0]], rows_v.at[p], gsem[p]).wait()

        pltpu.sync_copy(zero_hbm, rows_v.at[0])
        base = s * rpt
        nfull, tail = rpt // B, rpt % B
        for k in range(nfull):
            pltpu.sync_copy(rows_v.at[0], acc.at[pl.ds(base + k * B, B)])
        if tail:
            pltpu.sync_copy(rows_v.at[0, pl.ds(0, tail)],
                            acc.at[pl.ds(base + nfull * B, tail)])

        for k in range(RING):
            idx_start(k, k)
        for k in range(DEPTH):
            idx_wait(k)
        plsc.subcore_barrier()
        for k in range(DEPTH):
            gather_start(k, k)

        def body(g, carry):
            for k in range(RING):
                j = g * RING + k
                p = k % DEPTH
                kpd = (k + DEPTH) % RING
                gather_wait(p)
                pltpu.sync_copy(rows_v.at[p], acc.at[dstb.at[k]], add=True)
                idx_start(jnp.minimum(j + RING, nbc - 1), k)
                idx_wait(kpd)
                gather_start(kpd, p)
            return carry

        lax.fori_loop(0, nbc // RING, body, 0)

        for k in range(DEPTH):
            idx_wait((DEPTH + k) % RING)
            gather_wait(k)

        plsc.subcore_barrier()
        pltpu.sync_copy(acc.at[pl.ds(s * rpt, rpt)],
                        out_hbm.at[c, pl.ds(s * rpt, rpt)])

    return seg_sum(table, src_p, dst_p, zeros_b)


def _mlp_call(h, agg, Wa, ba, Wb, bb):
    n, D = h.shape
    blk = 2000
    grid = (n // blk,)

    def body(h_ref, a_ref, wa_ref, ba_ref, wb_ref, bb_ref, o_ref):
        r = h_ref[...] + a_ref[0] + a_ref[1]
        t = jnp.maximum(
            jnp.dot(r, wa_ref[...], preferred_element_type=jnp.float32)
            + ba_ref[...], 0.0)
        o_ref[...] = (jnp.dot(t, wb_ref[...], preferred_element_type=jnp.float32)
                      + bb_ref[...])

    return pl.pallas_call(
        body,
        grid=grid,
        in_specs=[
            pl.BlockSpec((blk, D), lambda i: (i, 0)),
            pl.BlockSpec((2, blk, D), lambda i: (0, i, 0)),
            pl.BlockSpec((D, D), lambda i: (0, 0)),
            pl.BlockSpec((1, D), lambda i: (0, 0)),
            pl.BlockSpec((D, D), lambda i: (0, 0)),
            pl.BlockSpec((1, D), lambda i: (0, 0)),
        ],
        out_specs=pl.BlockSpec((blk, D), lambda i: (i, 0)),
        out_shape=jax.ShapeDtypeStruct((n, D), jnp.float32),
    )(h, agg, Wa, ba, Wb, bb)


def _final_call(h, agg, W3, b3, W4, b4, Wl, bl):
    n, D = h.shape
    O = Wl.shape[1]
    blk = 2000
    grid = (n // blk,)

    def body(h_ref, a_ref, w3_ref, b3_ref, w4_ref, b4_ref, wl_ref, bl_ref,
             o_ref, acc_ref):
        i = pl.program_id(0)

        @pl.when(i == 0)
        def _():
            acc_ref[...] = jnp.zeros_like(acc_ref)

        r = h_ref[...] + a_ref[0] + a_ref[1]
        t = jnp.maximum(
            jnp.dot(r, w3_ref[...], preferred_element_type=jnp.float32)
            + b3_ref[...], 0.0)
        acc_ref[...] += jnp.sum(t, axis=0, keepdims=True)

        @pl.when(i == grid[0] - 1)
        def _():
            hg = acc_ref[...] * (1.0 / n)
            t2 = (jnp.dot(hg, w4_ref[...], preferred_element_type=jnp.float32)
                  + b4_ref[...])
            o_ref[...] = (jnp.dot(t2, wl_ref[...],
                                  preferred_element_type=jnp.float32)
                          + bl_ref[...])

    return pl.pallas_call(
        body,
        grid=grid,
        in_specs=[
            pl.BlockSpec((blk, D), lambda i: (i, 0)),
            pl.BlockSpec((2, blk, D), lambda i: (0, i, 0)),
            pl.BlockSpec((D, D), lambda i: (0, 0)),
            pl.BlockSpec((1, D), lambda i: (0, 0)),
            pl.BlockSpec((D, D), lambda i: (0, 0)),
            pl.BlockSpec((1, D), lambda i: (0, 0)),
            pl.BlockSpec((D, O), lambda i: (0, 0)),
            pl.BlockSpec((1, O), lambda i: (0, 0)),
        ],
        out_specs=pl.BlockSpec((1, O), lambda i: (0, 0)),
        out_shape=jax.ShapeDtypeStruct((1, O), jnp.float32),
        scratch_shapes=[pltpu.VMEM((1, D), jnp.float32)],
    )(h, agg, W3, b3, W4, b4, Wl, bl)


def kernel(features, edge_index, W1, b1, W2, b2, W3, b3, W4, b4, Wl, bl):
    n, D = features.shape
    E = edge_index.shape[1]

    n_pad = ((n + 1 + NS * 8 - 1) // (NS * 8)) * (NS * 8)
    per_b = NS * B
    nbt = -(-E // per_b)
    nbt = ((nbt + 2 * RING - 1) // (2 * RING)) * (2 * RING)
    nb0 = nbt - ((nbt * 39 // 100) // RING) * RING
    nb1 = nbt - nb0
    nbw = max(nb0, nb1)
    e_pad = nbt * per_b

    src = edge_index[0]
    dst = edge_index[1]
    pad = e_pad - E
    src_f = jnp.concatenate([src, jnp.zeros((pad,), jnp.int32)])
    dst_f = jnp.concatenate([dst, jnp.full((pad,), n, jnp.int32)])

    def split(flat):
        a = flat[:nb0 * per_b].reshape(NS, nb0, B)
        b = flat[nb0 * per_b:].reshape(NS, nb1, B)
        a = jnp.pad(a, ((0, 0), (0, nbw - nb0), (0, 0)))
        b = jnp.pad(b, ((0, 0), (0, nbw - nb1), (0, 0)))
        return jnp.stack([a, b])

    src_p = split(src_f)
    dst_p = split(dst_f)
    zeros_b = jnp.zeros((B, D), jnp.float32)

    b1r = b1.reshape(1, -1)
    b2r = b2.reshape(1, -1)
    b3r = b3.reshape(1, -1)
    b4r = b4.reshape(1, -1)
    blr = bl.reshape(1, -1)

    agg1 = _seg_sum_call(features, src_p, dst_p, zeros_b, n_pad, nb0, nb1)
    h1 = _mlp_call(features, agg1, W1, b1r, W2, b2r)
    agg2 = _seg_sum_call(h1, src_p, dst_p, zeros_b, n_pad, nb0, nb1)
    return _final_call(h1, agg2, W3, b3r, W4, b4r, Wl, blr)


# speedup vs baseline: 1.1759x; 1.0005x over previous
import functools

import jax
import jax.numpy as jnp
from jax import lax
from jax.experimental import pallas as pl
from jax.experimental.pallas import tpu as pltpu
from jax.experimental.pallas import tpu_sc as plsc

NC = 2
NS = 16
B = 88
DEPTH = 3
RING = 2 * DEPTH


def _seg_sum_call(table, src_p, dst_p, zeros_b, n_pad, nb0, nb1):
    D = table.shape[1]
    rpt = n_pad // NS

    mesh = plsc.VectorSubcoreMesh(core_axis_name="c", subcore_axis_name="s")

    @functools.partial(
        pl.kernel,
        out_type=jax.ShapeDtypeStruct((NC, n_pad, D), jnp.float32),
        mesh=mesh,
        scratch_types=[
            pltpu.VMEM_SHARED((n_pad, D), jnp.float32),
            pltpu.VMEM((RING, B), jnp.int32),
            pltpu.VMEM((RING, B), jnp.int32),
            pltpu.VMEM((DEPTH, B, D), jnp.float32),
            [pltpu.SemaphoreType.DMA] * DEPTH,
            [pltpu.SemaphoreType.DMA] * RING,
        ],
    )
    def seg_sum(table_hbm, src_hbm, dst_hbm, zero_hbm, out_hbm,
                acc, srcb, dstb, rows_v, gsem, semi):
        c = lax.axis_index("c")
        s = lax.axis_index("s")
        nbc = jnp.where(c == 0, nb0, nb1)

        def idx_start(batch, k):
            pltpu.async_copy(src_hbm.at[c, s, batch], srcb.at[k], semi[k])
            pltpu.async_copy(dst_hbm.at[c, s, batch], dstb.at[k], semi[k])

        def idx_wait(k):
            pltpu.make_async_copy(src_hbm.at[0, 0, 0], srcb.at[k], semi[k]).wait()
            pltpu.make_async_copy(src_hbm.at[0, 0, 0], dstb.at[k], semi[k]).wait()

        def gather_start(k, p):
            pltpu.async_copy(table_hbm.at[srcb.at[k]], rows_v.at[p], gsem[p])

        def gather_wait(p):
            pltpu.make_async_copy(
                table_hbm.at[srcb.at[0]], rows_v.at[p], gsem[p]).wait()

        pltpu.sync_copy(zero_hbm, rows_v.at[0])
        base = s * rpt
        nfull, tail = rpt // B, rpt % B
        for k in range(nfull):
            pltpu.sync_copy(rows_v.at[0], acc.at[pl.ds(base + k * B, B)])
        if tail:
            pltpu.sync_copy(rows_v.at[0, pl.ds(0, tail)],
                            acc.at[pl.ds(base + nfull * B, tail)])

        for k in range(RING):
            idx_start(k, k)
        for k in range(DEPTH):
            idx_wait(k)
        plsc.subcore_barrier()
        for k in range(DEPTH):
            gather_start(k, k)

        def body(g, carry):
            for k in range(RING):
                j = g * RING + k
                p = k % DEPTH
                kpd = (k + DEPTH) % RING
                gather_wait(p)
                pltpu.sync_copy(rows_v.at[p], acc.at[dstb.at[k]], add=True)
                idx_start(jnp.minimum(j + RING, nbc - 1), k)
                idx_wait(kpd)
                gather_start(kpd, p)
            return carry

        lax.fori_loop(0, nbc // RING, body, 0)

        for k in range(DEPTH):
            idx_wait((DEPTH + k) % RING)
            gather_wait(k)

        plsc.subcore_barrier()
        pltpu.sync_copy(acc.at[pl.ds(s * rpt, rpt)],
                        out_hbm.at[c, pl.ds(s * rpt, rpt)])

    return seg_sum(table, src_p, dst_p, zeros_b)


def _mlp_call(h, agg, Wa, ba, Wb, bb):
    n, D = h.shape
    blk = 2000
    grid = (n // blk,)

    def body(h_ref, a_ref, wa_ref, ba_ref, wb_ref, bb_ref, o_ref):
        r = h_ref[...] + a_ref[0] + a_ref[1]
        t = jnp.maximum(
            jnp.dot(r, wa_ref[...], preferred_element_type=jnp.float32)
            + ba_ref[...], 0.0)
        o_ref[...] = (jnp.dot(t, wb_ref[...], preferred_element_type=jnp.float32)
                      + bb_ref[...])

    return pl.pallas_call(
        body,
        grid=grid,
        in_specs=[
            pl.BlockSpec((blk, D), lambda i: (i, 0)),
            pl.BlockSpec((2, blk, D), lambda i: (0, i, 0)),
            pl.BlockSpec((D, D), lambda i: (0, 0)),
            pl.BlockSpec((1, D), lambda i: (0, 0)),
            pl.BlockSpec((D, D), lambda i: (0, 0)),
            pl.BlockSpec((1, D), lambda i: (0, 0)),
        ],
        out_specs=pl.BlockSpec((blk, D), lambda i: (i, 0)),
        out_shape=jax.ShapeDtypeStruct((n, D), jnp.float32),
    )(h, agg, Wa, ba, Wb, bb)


def _final_call(h, agg, W3, b3, W4, b4, Wl, bl):
    n, D = h.shape
    O = Wl.shape[1]
    blk = 2000
    grid = (n // blk,)

    def body(h_ref, a_ref, w3_ref, b3_ref, w4_ref, b4_ref, wl_ref, bl_ref,
             o_ref, acc_ref):
        i = pl.program_id(0)

        @pl.when(i == 0)
        def _():
            acc_ref[...] = jnp.zeros_like(acc_ref)

        r = h_ref[...] + a_ref[0] + a_ref[1]
        t = jnp.maximum(
            jnp.dot(r, w3_ref[...], preferred_element_type=jnp.float32)
            + b3_ref[...], 0.0)
        acc_ref[...] += jnp.sum(t, axis=0, keepdims=True)

        @pl.when(i == grid[0] - 1)
        def _():
            hg = acc_ref[...] * (1.0 / n)
            t2 = (jnp.dot(hg, w4_ref[...], preferred_element_type=jnp.float32)
                  + b4_ref[...])
            o_ref[...] = (jnp.dot(t2, wl_ref[...],
                                  preferred_element_type=jnp.float32)
                          + bl_ref[...])

    return pl.pallas_call(
        body,
        grid=grid,
        in_specs=[
            pl.BlockSpec((blk, D), lambda i: (i, 0)),
            pl.BlockSpec((2, blk, D), lambda i: (0, i, 0)),
            pl.BlockSpec((D, D), lambda i: (0, 0)),
            pl.BlockSpec((1, D), lambda i: (0, 0)),
            pl.BlockSpec((D, D), lambda i: (0, 0)),
            pl.BlockSpec((1, D), lambda i: (0, 0)),
            pl.BlockSpec((D, O), lambda i: (0, 0)),
            pl.BlockSpec((1, O), lambda i: (0, 0)),
        ],
        out_specs=pl.BlockSpec((1, O), lambda i: (0, 0)),
        out_shape=jax.ShapeDtypeStruct((1, O), jnp.float32),
        scratch_shapes=[pltpu.VMEM((1, D), jnp.float32)],
    )(h, agg, W3, b3, W4, b4, Wl, bl)


def kernel(features, edge_index, W1, b1, W2, b2, W3, b3, W4, b4, Wl, bl):
    n, D = features.shape
    E = edge_index.shape[1]

    n_pad = ((n + 1 + NS * 8 - 1) // (NS * 8)) * (NS * 8)
    per_b = NS * B
    nbt = -(-E // per_b)
    nbt = ((nbt + 2 * RING - 1) // (2 * RING)) * (2 * RING)
    nb0 = nbt - ((nbt * 45 // 100) // RING) * RING
    nb1 = nbt - nb0
    nbw = max(nb0, nb1)
    e_pad = nbt * per_b

    src = edge_index[0]
    dst = edge_index[1]
    pad = e_pad - E
    src_f = jnp.concatenate([src, jnp.zeros((pad,), jnp.int32)])
    dst_f = jnp.concatenate([dst, jnp.full((pad,), n, jnp.int32)])

    def split(flat):
        a = flat[:nb0 * per_b].reshape(NS, nb0, B)
        b = flat[nb0 * per_b:].reshape(NS, nb1, B)
        a = jnp.pad(a, ((0, 0), (0, nbw - nb0), (0, 0)))
        b = jnp.pad(b, ((0, 0), (0, nbw - nb1), (0, 0)))
        return jnp.stack([a, b])

    src_p = split(src_f)
    dst_p = split(dst_f)
    zeros_b = jnp.zeros((B, D), jnp.float32)

    b1r = b1.reshape(1, -1)
    b2r = b2.reshape(1, -1)
    b3r = b3.reshape(1, -1)
    b4r = b4.reshape(1, -1)
    blr = bl.reshape(1, -1)

    agg1 = _seg_sum_call(features, src_p, dst_p, zeros_b, n_pad, nb0, nb1)
    h1 = _mlp_call(features, agg1, W1, b1r, W2, b2r)
    agg2 = _seg_sum_call(h1, src_p, dst_p, zeros_b, n_pad, nb0, nb1)
    return _final_call(h1, agg2, W3, b3r, W4, b4r, Wl, blr)
